# baseline (speedup 1.0000x reference)
import functools

import jax
import jax.numpy as jnp
from jax import lax
from jax.experimental import pallas as pl
from jax.experimental.pallas import tpu as pltpu
from jax.experimental.pallas import tpu_sc as plsc

N_NODES = 10000
N_EDGES = 320000
D_FEAT = 128
D_EDGE = 16

_NW = 32
_SPLITS = 1
_NSLOT = 4


_TC1_BLOCK = 1000


def _pack_bf16(x):
    bf = x.astype(jnp.bfloat16)
    lo = lax.bitcast_convert_type(bf[:, :64], jnp.uint16).astype(jnp.uint32)
    hi = lax.bitcast_convert_type(bf[:, 64:], jnp.uint16).astype(jnp.uint32)
    return lax.bitcast_convert_type(lo | (hi << 16), jnp.int32)


def _tc1_body(vd_ref, ws_ref, wr_ref, ps_ref, pr_ref):
    vd = vd_ref[...]
    ps = jnp.dot(vd, ws_ref[...], preferred_element_type=jnp.float32)
    pr = jnp.dot(vd, wr_ref[...], preferred_element_type=jnp.float32)
    ps_ref[...] = _pack_bf16(ps)
    pr_ref[...] = _pack_bf16(pr)


def _node_projections(vdata, ws_t, wr_t):
    grid = N_NODES // _TC1_BLOCK
    return pl.pallas_call(
        _tc1_body,
        grid=(grid,),
        in_specs=[
            pl.BlockSpec((_TC1_BLOCK, D_FEAT), lambda i: (i, 0)),
            pl.BlockSpec((D_FEAT, D_FEAT), lambda i: (0, 0)),
            pl.BlockSpec((D_FEAT, D_FEAT), lambda i: (0, 0)),
        ],
        out_specs=[
            pl.BlockSpec((_TC1_BLOCK, D_FEAT // 2), lambda i: (i, 0)),
            pl.BlockSpec((_TC1_BLOCK, D_FEAT // 2), lambda i: (i, 0)),
        ],
        out_shape=[
            jax.ShapeDtypeStruct((N_NODES, D_FEAT // 2), jnp.int32),
            jax.ShapeDtypeStruct((N_NODES, D_FEAT // 2), jnp.int32),
        ],
    )(vdata, ws_t, wr_t)


def _chunking(epw):
    for c in range(128, 0, -8):
        full = epw // c
        if full >= _NSLOT and epw - full * c <= c:
            return c, full, epw - full * c
    raise ValueError(epw)


def _sc_gather_sum(senders, receivers, ps, pr, ne):
    epw = ne // _NW
    c, full, tail = _chunking(epw)
    nw64 = D_FEAT // 2
    mesh = plsc.VectorSubcoreMesh(core_axis_name="c", subcore_axis_name="s")

    @functools.partial(
        pl.kernel,
        mesh=mesh,
        out_type=[
            jax.ShapeDtypeStruct((ne, nw64), jnp.int32),
            jax.ShapeDtypeStruct((ne, nw64), jnp.int32),
        ],
        scratch_types=[
            pltpu.VMEM((epw,), jnp.int32),
            pltpu.VMEM((epw,), jnp.int32),
        ] + [pltpu.VMEM((c, nw64), jnp.int32)] * (2 * _NSLOT)
          + [pltpu.SemaphoreType.DMA] * (2 * _NSLOT),
        compiler_params=pltpu.CompilerParams(use_tc_tiling_on_sc=False),
    )
    def k(sidx_hbm, ridx_hbm, ps_hbm, pr_hbm, outs_hbm, outr_hbm,
          sidx_v, ridx_v, *bufs):
        rows_s = bufs[:_NSLOT]
        rows_r = bufs[_NSLOT:2 * _NSLOT]
        semg_v = bufs[2 * _NSLOT:3 * _NSLOT]
        semw_v = bufs[3 * _NSLOT:]
        wid = lax.axis_index("s") * 2 + lax.axis_index("c")
        base = wid * epw
        pltpu.sync_copy(sidx_hbm.at[pl.ds(base, epw)], sidx_v)
        pltpu.sync_copy(ridx_hbm.at[pl.ds(base, epw)], ridx_v)

        def gathers(ci, b):
            pltpu.async_copy(ps_hbm.at[sidx_v.at[pl.ds(ci * c, c)]],
                             rows_s[b], semg_v[b])
            pltpu.async_copy(pr_hbm.at[ridx_v.at[pl.ds(ci * c, c)]],
                             rows_r[b], semg_v[b])

        def wait_gathers(b):
            pltpu.make_async_copy(ps_hbm.at[pl.ds(0, c)], rows_s[b],
                                  semg_v[b]).wait()
            pltpu.make_async_copy(pr_hbm.at[pl.ds(0, c)], rows_r[b],
                                  semg_v[b]).wait()

        def writes(ci, b):
            pltpu.async_copy(rows_s[b], outs_hbm.at[pl.ds(base + ci * c, c)],
                             semw_v[b])
            pltpu.async_copy(rows_r[b], outr_hbm.at[pl.ds(base + ci * c, c)],
                             semw_v[b])

        def wait_writes(b):
            pltpu.make_async_copy(rows_s[b], outs_hbm.at[pl.ds(0, c)],
                                  semw_v[b]).wait()
            pltpu.make_async_copy(rows_r[b], outr_hbm.at[pl.ds(0, c)],
                                  semw_v[b]).wait()

        for b in range(_NSLOT):
            gathers(b, b)

        def ring_body(j, carry):
            for b in range(_NSLOT):
                ci = _NSLOT * j + b

                @pl.when(ci < full)
                def _():
                    wait_gathers(b)
                    writes(ci, b)

                    @pl.when(ci + _NSLOT < full)
                    def _():
                        wait_writes(b)
                        gathers(ci + _NSLOT, b)

            return carry

        lax.fori_loop(0, (full + _NSLOT - 1) // _NSLOT, ring_body, 0)

        for b in range(_NSLOT):
            wait_writes(b)

        if tail:
            toff = full * c
            rs_t = rows_s[0].at[pl.ds(0, tail)]
            rr_t = rows_r[0].at[pl.ds(0, tail)]
            pltpu.async_copy(
                ps_hbm.at[sidx_v.at[pl.ds(toff, tail)]], rs_t, semg_v[0])
            pltpu.async_copy(
                pr_hbm.at[ridx_v.at[pl.ds(toff, tail)]], rr_t, semg_v[0])
            pltpu.make_async_copy(ps_hbm.at[pl.ds(0, tail)], rs_t,
                                  semg_v[0]).wait()
            pltpu.make_async_copy(pr_hbm.at[pl.ds(0, tail)], rr_t,
                                  semg_v[0]).wait()
            pltpu.sync_copy(rs_t, outs_hbm.at[pl.ds(base + toff, tail)])
            pltpu.sync_copy(rr_t, outr_hbm.at[pl.ds(base + toff, tail)])

    return k(senders, receivers, ps, pr)


_TC2_BLOCK = 4000


def _unpack_bf16(w32):
    lo = lax.bitcast_convert_type(w32 << 16, jnp.float32)
    hi = lax.bitcast_convert_type(w32 & jnp.int32(-65536), jnp.float32)
    return jnp.concatenate([lo, hi], axis=1)


def _tc2_body(gs_ref, gr_ref, ed_ref, we_ref, b_ref, out_ref):
    prod = jnp.dot(ed_ref[...], we_ref[...], preferred_element_type=jnp.float32)
    g = _unpack_bf16(gs_ref[...]) + _unpack_bf16(gr_ref[...])
    out_ref[...] = g + prod + b_ref[...]


def _edge_update(gathered, edata, we_t, b2d, ne):
    gs, gr = gathered
    grid = ne // _TC2_BLOCK
    return pl.pallas_call(
        _tc2_body,
        grid=(grid,),
        in_specs=[
            pl.BlockSpec((_TC2_BLOCK, D_FEAT // 2), lambda i: (i, 0)),
            pl.BlockSpec((_TC2_BLOCK, D_FEAT // 2), lambda i: (i, 0)),
            pl.BlockSpec((_TC2_BLOCK, D_EDGE), lambda i: (i, 0)),
            pl.BlockSpec((D_EDGE, D_FEAT), lambda i: (0, 0)),
            pl.BlockSpec((1, D_FEAT), lambda i: (0, 0)),
        ],
        out_specs=pl.BlockSpec((_TC2_BLOCK, D_FEAT), lambda i: (i, 0)),
        out_shape=jax.ShapeDtypeStruct((ne, D_FEAT), jnp.float32),
    )(gs, gr, edata, we_t, b2d)


def kernel(vdata, edata, connectivity, W, b):
    senders = connectivity[0].astype(jnp.int32)
    receivers = connectivity[1].astype(jnp.int32)
    we_t = W[:, :D_EDGE].T
    ws_t = W[:, D_EDGE:D_EDGE + D_FEAT].T
    wr_t = W[:, D_EDGE + D_FEAT:].T
    b2d = b.reshape(1, D_FEAT)
    ps, pr = _node_projections(vdata, ws_t, wr_t)

    h = N_EDGES // _SPLITS
    outs = []
    for p in range(_SPLITS):
        sl = slice(p * h, (p + 1) * h)
        g = _sc_gather_sum(senders[sl], receivers[sl], ps, pr, h)
        outs.append(_edge_update(g, edata[sl], we_t, b2d, h))
    if _SPLITS == 1:
        return outs[0]
    return jnp.concatenate(outs, axis=0)


# speedup vs baseline: 1.6899x; 1.6899x over previous
import functools

import jax
import jax.numpy as jnp
from jax import lax
from jax.experimental import pallas as pl
from jax.experimental.pallas import tpu as pltpu
from jax.experimental.pallas import tpu_sc as plsc

N_NODES = 10000
N_EDGES = 320000
D_FEAT = 128
D_EDGE = 16

_NW = 32
_SPLITS = 1
_NSLOT = 4


_TC1_BLOCK = 1000


def _pack_bf16(x):
    bf = x.astype(jnp.bfloat16)
    lo = lax.bitcast_convert_type(bf[:, :64], jnp.uint16).astype(jnp.uint32)
    hi = lax.bitcast_convert_type(bf[:, 64:], jnp.uint16).astype(jnp.uint32)
    return lax.bitcast_convert_type(lo | (hi << 16), jnp.int32)


def _tc1_body(vd_ref, ws_ref, wr_ref, ps_ref, pr_ref):
    vd = vd_ref[...]
    ps = jnp.dot(vd, ws_ref[...], preferred_element_type=jnp.float32)
    pr = jnp.dot(vd, wr_ref[...], preferred_element_type=jnp.float32)
    ps_ref[...] = _pack_bf16(ps)
    pr_ref[...] = _pack_bf16(pr)


def _node_projections(vdata, ws_t, wr_t):
    grid = N_NODES // _TC1_BLOCK
    return pl.pallas_call(
        _tc1_body,
        grid=(grid,),
        in_specs=[
            pl.BlockSpec((_TC1_BLOCK, D_FEAT), lambda i: (i, 0)),
            pl.BlockSpec((D_FEAT, D_FEAT), lambda i: (0, 0)),
            pl.BlockSpec((D_FEAT, D_FEAT), lambda i: (0, 0)),
        ],
        out_specs=[
            pl.BlockSpec((_TC1_BLOCK, D_FEAT // 2), lambda i: (i, 0)),
            pl.BlockSpec((_TC1_BLOCK, D_FEAT // 2), lambda i: (i, 0)),
        ],
        out_shape=[
            jax.ShapeDtypeStruct((N_NODES, D_FEAT // 2), jnp.int32),
            jax.ShapeDtypeStruct((N_NODES, D_FEAT // 2), jnp.int32),
        ],
    )(vdata, ws_t, wr_t)


def _chunking(epw):
    for c in range(128, 0, -8):
        full = epw // c
        if full >= _NSLOT and epw - full * c <= c:
            return c, full, epw - full * c
    raise ValueError(epw)


def _sc_gather_sum(senders, receivers, ps, pr, ne):
    epw = ne // _NW
    c, full, tail = _chunking(epw)
    nw64 = D_FEAT // 2
    mesh = plsc.VectorSubcoreMesh(core_axis_name="c", subcore_axis_name="s")

    @functools.partial(
        pl.kernel,
        mesh=mesh,
        out_type=jax.ShapeDtypeStruct((ne, D_FEAT), jnp.int32),
        scratch_types=[
            pltpu.VMEM((epw,), jnp.int32),
            pltpu.VMEM((epw,), jnp.int32),
        ] + [pltpu.VMEM((c, nw64), jnp.int32)] * (2 * _NSLOT)
          + [pltpu.SemaphoreType.DMA] * (2 * _NSLOT),
        compiler_params=pltpu.CompilerParams(use_tc_tiling_on_sc=False),
    )
    def k(sidx_hbm, ridx_hbm, ps_hbm, pr_hbm, out_hbm,
          sidx_v, ridx_v, *bufs):
        rows_s = bufs[:_NSLOT]
        rows_r = bufs[_NSLOT:2 * _NSLOT]
        semg_v = bufs[2 * _NSLOT:3 * _NSLOT]
        semw_v = bufs[3 * _NSLOT:]
        wid = lax.axis_index("s") * 2 + lax.axis_index("c")
        base = wid * epw
        pltpu.sync_copy(sidx_hbm.at[pl.ds(base, epw)], sidx_v)
        pltpu.sync_copy(ridx_hbm.at[pl.ds(base, epw)], ridx_v)

        def gathers(ci, b):
            pltpu.async_copy(ps_hbm.at[sidx_v.at[pl.ds(ci * c, c)]],
                             rows_s[b], semg_v[b])
            pltpu.async_copy(pr_hbm.at[ridx_v.at[pl.ds(ci * c, c)]],
                             rows_r[b], semg_v[b])

        def wait_gathers(b):
            pltpu.make_async_copy(ps_hbm.at[pl.ds(0, c)], rows_s[b],
                                  semg_v[b]).wait()
            pltpu.make_async_copy(pr_hbm.at[pl.ds(0, c)], rows_r[b],
                                  semg_v[b]).wait()

        def writes(ci, b):
            dst = out_hbm.at[pl.ds(base + ci * c, c)]
            pltpu.async_copy(rows_s[b], dst.at[:, pl.ds(0, nw64)], semw_v[b])
            pltpu.async_copy(rows_r[b], dst.at[:, pl.ds(nw64, nw64)],
                             semw_v[b])

        def wait_writes(b):
            pltpu.make_async_copy(
                rows_s[b], out_hbm.at[pl.ds(0, c), pl.ds(0, nw64)],
                semw_v[b]).wait()
            pltpu.make_async_copy(
                rows_r[b], out_hbm.at[pl.ds(0, c), pl.ds(nw64, nw64)],
                semw_v[b]).wait()

        for b in range(_NSLOT):
            gathers(b, b)

        def ring_body(j, carry):
            for b in range(_NSLOT):
                ci = _NSLOT * j + b

                @pl.when(ci < full)
                def _():
                    wait_gathers(b)
                    writes(ci, b)

                    @pl.when(ci + _NSLOT < full)
                    def _():
                        wait_writes(b)
                        gathers(ci + _NSLOT, b)

            return carry

        lax.fori_loop(0, (full + _NSLOT - 1) // _NSLOT, ring_body, 0)

        for b in range(_NSLOT):
            wait_writes(b)

        if tail:
            toff = full * c
            rs_t = rows_s[0].at[pl.ds(0, tail)]
            rr_t = rows_r[0].at[pl.ds(0, tail)]
            pltpu.async_copy(
                ps_hbm.at[sidx_v.at[pl.ds(toff, tail)]], rs_t, semg_v[0])
            pltpu.async_copy(
                pr_hbm.at[ridx_v.at[pl.ds(toff, tail)]], rr_t, semg_v[0])
            pltpu.make_async_copy(ps_hbm.at[pl.ds(0, tail)], rs_t,
                                  semg_v[0]).wait()
            pltpu.make_async_copy(pr_hbm.at[pl.ds(0, tail)], rr_t,
                                  semg_v[0]).wait()
            tdst = out_hbm.at[pl.ds(base + toff, tail)]
            pltpu.sync_copy(rs_t, tdst.at[:, pl.ds(0, nw64)])
            pltpu.sync_copy(rr_t, tdst.at[:, pl.ds(nw64, nw64)])

    return k(senders, receivers, ps, pr)


_TC2_BLOCK = 4000


def _unpack_bf16(w32):
    lo = lax.bitcast_convert_type(w32 << 16, jnp.float32)
    hi = lax.bitcast_convert_type(w32 & jnp.int32(-65536), jnp.float32)
    return jnp.concatenate([lo, hi], axis=1)


def _tc2_body(g_ref, ed_ref, we_ref, b_ref, out_ref):
    prod = jnp.dot(ed_ref[...], we_ref[...], preferred_element_type=jnp.float32)
    g32 = g_ref[...]
    g = _unpack_bf16(g32[:, :D_FEAT // 2]) + _unpack_bf16(g32[:, D_FEAT // 2:])
    out_ref[...] = g + prod + b_ref[...]


def _edge_update(gathered, edata, we_t, b2d, ne):
    grid = ne // _TC2_BLOCK
    return pl.pallas_call(
        _tc2_body,
        grid=(grid,),
        in_specs=[
            pl.BlockSpec((_TC2_BLOCK, D_FEAT), lambda i: (i, 0)),
            pl.BlockSpec((_TC2_BLOCK, D_EDGE), lambda i: (i, 0)),
            pl.BlockSpec((D_EDGE, D_FEAT), lambda i: (0, 0)),
            pl.BlockSpec((1, D_FEAT), lambda i: (0, 0)),
        ],
        out_specs=pl.BlockSpec((_TC2_BLOCK, D_FEAT), lambda i: (i, 0)),
        out_shape=jax.ShapeDtypeStruct((ne, D_FEAT), jnp.float32),
    )(gathered, edata, we_t, b2d)


def kernel(vdata, edata, connectivity, W, b):
    senders = connectivity[0].astype(jnp.int32)
    receivers = connectivity[1].astype(jnp.int32)
    we_t = W[:, :D_EDGE].T
    ws_t = W[:, D_EDGE:D_EDGE + D_FEAT].T
    wr_t = W[:, D_EDGE + D_FEAT:].T
    b2d = b.reshape(1, D_FEAT)
    ps, pr = _node_projections(vdata, ws_t, wr_t)

    h = N_EDGES // _SPLITS
    outs = []
    for p in range(_SPLITS):
        sl = slice(p * h, (p + 1) * h)
        g = _sc_gather_sum(senders[sl], receivers[sl], ps, pr, h)
        outs.append(_edge_update(g, edata[sl], we_t, b2d, h))
    if _SPLITS == 1:
        return outs[0]
    return jnp.concatenate(outs, axis=0)


# speedup vs baseline: 1.7412x; 1.0304x over previous
import functools

import jax
import jax.numpy as jnp
from jax import lax
from jax.experimental import pallas as pl
from jax.experimental.pallas import tpu as pltpu
from jax.experimental.pallas import tpu_sc as plsc

N_NODES = 10000
N_EDGES = 320000
D_FEAT = 128
D_EDGE = 16

_NW = 32
_SPLITS = 1
_NSLOT = 4


_TC1_BLOCK = 1000


def _pack_bf16(x):
    bf = x.astype(jnp.bfloat16)
    lo = lax.bitcast_convert_type(bf[:, :64], jnp.uint16).astype(jnp.uint32)
    hi = lax.bitcast_convert_type(bf[:, 64:], jnp.uint16).astype(jnp.uint32)
    return lax.bitcast_convert_type(lo | (hi << 16), jnp.int32)


def _tc1_body(vd_ref, ws_ref, wr_ref, ps_ref, pr_ref):
    vd = vd_ref[...]
    ps = jnp.dot(vd, ws_ref[...], preferred_element_type=jnp.float32)
    pr = jnp.dot(vd, wr_ref[...], preferred_element_type=jnp.float32)
    ps_ref[...] = _pack_bf16(ps)
    pr_ref[...] = _pack_bf16(pr)


def _node_projections(vdata, ws_t, wr_t):
    grid = N_NODES // _TC1_BLOCK
    return pl.pallas_call(
        _tc1_body,
        grid=(grid,),
        in_specs=[
            pl.BlockSpec((_TC1_BLOCK, D_FEAT), lambda i: (i, 0)),
            pl.BlockSpec((D_FEAT, D_FEAT), lambda i: (0, 0)),
            pl.BlockSpec((D_FEAT, D_FEAT), lambda i: (0, 0)),
        ],
        out_specs=[
            pl.BlockSpec((_TC1_BLOCK, D_FEAT // 2), lambda i: (i, 0)),
            pl.BlockSpec((_TC1_BLOCK, D_FEAT // 2), lambda i: (i, 0)),
        ],
        out_shape=[
            jax.ShapeDtypeStruct((N_NODES, D_FEAT // 2), jnp.int32),
            jax.ShapeDtypeStruct((N_NODES, D_FEAT // 2), jnp.int32),
        ],
    )(vdata, ws_t, wr_t)


def _chunking(epw):
    for c in range(128, 0, -8):
        full = epw // c
        if full >= _NSLOT and epw - full * c <= c:
            return c, full, epw - full * c
    raise ValueError(epw)


def _sc_gather_sum(senders, receivers, ps, pr, ne):
    epw = ne // _NW
    c, full, tail = _chunking(epw)
    nw64 = D_FEAT // 2
    mesh = plsc.VectorSubcoreMesh(core_axis_name="c", subcore_axis_name="s")

    @functools.partial(
        pl.kernel,
        mesh=mesh,
        out_type=jax.ShapeDtypeStruct((ne, D_FEAT), jnp.int32),
        scratch_types=[
            pltpu.VMEM((epw,), jnp.int32),
            pltpu.VMEM((epw,), jnp.int32),
        ] + [pltpu.VMEM((c, nw64), jnp.int32)] * (2 * _NSLOT)
          + [pltpu.SemaphoreType.DMA] * (2 * _NSLOT),
        compiler_params=pltpu.CompilerParams(use_tc_tiling_on_sc=False),
    )
    def k(sidx_hbm, ridx_hbm, ps_hbm, pr_hbm, out_hbm,
          sidx_v, ridx_v, *bufs):
        rows_s = bufs[:_NSLOT]
        rows_r = bufs[_NSLOT:2 * _NSLOT]
        semg_v = bufs[2 * _NSLOT:3 * _NSLOT]
        semw_v = bufs[3 * _NSLOT:]
        wid = lax.axis_index("s") * 2 + lax.axis_index("c")
        base = wid * epw
        pltpu.sync_copy(sidx_hbm.at[pl.ds(base, epw)], sidx_v)
        pltpu.sync_copy(ridx_hbm.at[pl.ds(base, epw)], ridx_v)

        def gathers(ci, b):
            pltpu.async_copy(ps_hbm.at[sidx_v.at[pl.ds(ci * c, c)]],
                             rows_s[b], semg_v[b])
            pltpu.async_copy(pr_hbm.at[ridx_v.at[pl.ds(ci * c, c)]],
                             rows_r[b], semg_v[b])

        def wait_gathers(b):
            pltpu.make_async_copy(ps_hbm.at[pl.ds(0, c)], rows_s[b],
                                  semg_v[b]).wait()
            pltpu.make_async_copy(pr_hbm.at[pl.ds(0, c)], rows_r[b],
                                  semg_v[b]).wait()

        def writes(ci, b):
            dst = out_hbm.at[pl.ds(base + ci * c, c)]
            pltpu.async_copy(rows_s[b], dst.at[:, pl.ds(0, nw64)], semw_v[b])
            pltpu.async_copy(rows_r[b], dst.at[:, pl.ds(nw64, nw64)],
                             semw_v[b])

        def wait_writes(b):
            pltpu.make_async_copy(
                rows_s[b], out_hbm.at[pl.ds(0, c), pl.ds(0, nw64)],
                semw_v[b]).wait()
            pltpu.make_async_copy(
                rows_r[b], out_hbm.at[pl.ds(0, c), pl.ds(nw64, nw64)],
                semw_v[b]).wait()

        for b in range(_NSLOT):
            gathers(b, b)

        def ring_body(j, carry):
            for b in range(_NSLOT):
                ci = _NSLOT * j + b

                @pl.when(ci < full)
                def _():
                    wait_gathers(b)
                    writes(ci, b)

                    @pl.when(ci + _NSLOT < full)
                    def _():
                        wait_writes(b)
                        gathers(ci + _NSLOT, b)

            return carry

        lax.fori_loop(0, (full + _NSLOT - 1) // _NSLOT, ring_body, 0)

        for b in range(_NSLOT):
            wait_writes(b)

        if tail:
            toff = full * c
            rs_t = rows_s[0].at[pl.ds(0, tail)]
            rr_t = rows_r[0].at[pl.ds(0, tail)]
            pltpu.async_copy(
                ps_hbm.at[sidx_v.at[pl.ds(toff, tail)]], rs_t, semg_v[0])
            pltpu.async_copy(
                pr_hbm.at[ridx_v.at[pl.ds(toff, tail)]], rr_t, semg_v[0])
            pltpu.make_async_copy(ps_hbm.at[pl.ds(0, tail)], rs_t,
                                  semg_v[0]).wait()
            pltpu.make_async_copy(pr_hbm.at[pl.ds(0, tail)], rr_t,
                                  semg_v[0]).wait()
            tdst = out_hbm.at[pl.ds(base + toff, tail)]
            pltpu.sync_copy(rs_t, tdst.at[:, pl.ds(0, nw64)])
            pltpu.sync_copy(rr_t, tdst.at[:, pl.ds(nw64, nw64)])

    return k(senders, receivers, ps, pr)


_TC2_BLOCK = 8000


def _unpack_bf16(w32):
    lo = lax.bitcast_convert_type(w32 << 16, jnp.float32)
    hi = lax.bitcast_convert_type(w32 & jnp.int32(-65536), jnp.float32)
    return jnp.concatenate([lo, hi], axis=1)


def _tc2_body(g_ref, ed_ref, we_ref, b_ref, out_ref):
    prod = jnp.dot(ed_ref[...], we_ref[...], preferred_element_type=jnp.float32)
    g32 = g_ref[...]
    g = _unpack_bf16(g32[:, :D_FEAT // 2]) + _unpack_bf16(g32[:, D_FEAT // 2:])
    out_ref[...] = g + prod + b_ref[...]


def _edge_update(gathered, edata, we_t, b2d, ne):
    grid = ne // _TC2_BLOCK
    return pl.pallas_call(
        _tc2_body,
        grid=(grid,),
        in_specs=[
            pl.BlockSpec((_TC2_BLOCK, D_FEAT), lambda i: (i, 0)),
            pl.BlockSpec((_TC2_BLOCK, D_EDGE), lambda i: (i, 0)),
            pl.BlockSpec((D_EDGE, D_FEAT), lambda i: (0, 0)),
            pl.BlockSpec((1, D_FEAT), lambda i: (0, 0)),
        ],
        out_specs=pl.BlockSpec((_TC2_BLOCK, D_FEAT), lambda i: (i, 0)),
        out_shape=jax.ShapeDtypeStruct((ne, D_FEAT), jnp.float32),
    )(gathered, edata, we_t, b2d)


def kernel(vdata, edata, connectivity, W, b):
    senders = connectivity[0].astype(jnp.int32)
    receivers = connectivity[1].astype(jnp.int32)
    we_t = W[:, :D_EDGE].T
    ws_t = W[:, D_EDGE:D_EDGE + D_FEAT].T
    wr_t = W[:, D_EDGE + D_FEAT:].T
    b2d = b.reshape(1, D_FEAT)
    ps, pr = _node_projections(vdata, ws_t, wr_t)

    h = N_EDGES // _SPLITS
    outs = []
    for p in range(_SPLITS):
        sl = slice(p * h, (p + 1) * h)
        g = _sc_gather_sum(senders[sl], receivers[sl], ps, pr, h)
        outs.append(_edge_update(g, edata[sl], we_t, b2d, h))
    if _SPLITS == 1:
        return outs[0]
    return jnp.concatenate(outs, axis=0)


# speedup vs baseline: 1.7421x; 1.0005x over previous
import functools

import jax
import jax.numpy as jnp
from jax import lax
from jax.experimental import pallas as pl
from jax.experimental.pallas import tpu as pltpu
from jax.experimental.pallas import tpu_sc as plsc

N_NODES = 10000
N_EDGES = 320000
D_FEAT = 128
D_EDGE = 16

_NW = 32
_SPLITS = 1
_NSLOT = 4


_TC1_BLOCK = 2000


def _pack_bf16(x):
    bf = x.astype(jnp.bfloat16)
    lo = lax.bitcast_convert_type(bf[:, :64], jnp.uint16).astype(jnp.uint32)
    hi = lax.bitcast_convert_type(bf[:, 64:], jnp.uint16).astype(jnp.uint32)
    return lax.bitcast_convert_type(lo | (hi << 16), jnp.int32)


def _tc1_body(vd_ref, ws_ref, wr_ref, ps_ref, pr_ref):
    vd = vd_ref[...]
    ps = jnp.dot(vd, ws_ref[...], preferred_element_type=jnp.float32)
    pr = jnp.dot(vd, wr_ref[...], preferred_element_type=jnp.float32)
    ps_ref[...] = _pack_bf16(ps)
    pr_ref[...] = _pack_bf16(pr)


def _node_projections(vdata, ws_t, wr_t):
    grid = N_NODES // _TC1_BLOCK
    return pl.pallas_call(
        _tc1_body,
        grid=(grid,),
        in_specs=[
            pl.BlockSpec((_TC1_BLOCK, D_FEAT), lambda i: (i, 0)),
            pl.BlockSpec((D_FEAT, D_FEAT), lambda i: (0, 0)),
            pl.BlockSpec((D_FEAT, D_FEAT), lambda i: (0, 0)),
        ],
        out_specs=[
            pl.BlockSpec((_TC1_BLOCK, D_FEAT // 2), lambda i: (i, 0)),
            pl.BlockSpec((_TC1_BLOCK, D_FEAT // 2), lambda i: (i, 0)),
        ],
        out_shape=[
            jax.ShapeDtypeStruct((N_NODES, D_FEAT // 2), jnp.int32),
            jax.ShapeDtypeStruct((N_NODES, D_FEAT // 2), jnp.int32),
        ],
    )(vdata, ws_t, wr_t)


def _chunking(epw):
    for c in range(128, 0, -8):
        full = epw // c
        if full >= _NSLOT and epw - full * c <= c:
            return c, full, epw - full * c
    raise ValueError(epw)


def _sc_gather_sum(senders, receivers, ps, pr, ne):
    epw = ne // _NW
    c, full, tail = _chunking(epw)
    nw64 = D_FEAT // 2
    mesh = plsc.VectorSubcoreMesh(core_axis_name="c", subcore_axis_name="s")

    @functools.partial(
        pl.kernel,
        mesh=mesh,
        out_type=jax.ShapeDtypeStruct((ne, D_FEAT), jnp.int32),
        scratch_types=[
            pltpu.VMEM((epw,), jnp.int32),
            pltpu.VMEM((epw,), jnp.int32),
        ] + [pltpu.VMEM((c, nw64), jnp.int32)] * (2 * _NSLOT)
          + [pltpu.SemaphoreType.DMA] * (2 * _NSLOT),
        compiler_params=pltpu.CompilerParams(use_tc_tiling_on_sc=False),
    )
    def k(sidx_hbm, ridx_hbm, ps_hbm, pr_hbm, out_hbm,
          sidx_v, ridx_v, *bufs):
        rows_s = bufs[:_NSLOT]
        rows_r = bufs[_NSLOT:2 * _NSLOT]
        semg_v = bufs[2 * _NSLOT:3 * _NSLOT]
        semw_v = bufs[3 * _NSLOT:]
        wid = lax.axis_index("s") * 2 + lax.axis_index("c")
        base = wid * epw
        pltpu.sync_copy(sidx_hbm.at[pl.ds(base, epw)], sidx_v)
        pltpu.sync_copy(ridx_hbm.at[pl.ds(base, epw)], ridx_v)

        def gathers(ci, b):
            pltpu.async_copy(ps_hbm.at[sidx_v.at[pl.ds(ci * c, c)]],
                             rows_s[b], semg_v[b])
            pltpu.async_copy(pr_hbm.at[ridx_v.at[pl.ds(ci * c, c)]],
                             rows_r[b], semg_v[b])

        def wait_gathers(b):
            pltpu.make_async_copy(ps_hbm.at[pl.ds(0, c)], rows_s[b],
                                  semg_v[b]).wait()
            pltpu.make_async_copy(pr_hbm.at[pl.ds(0, c)], rows_r[b],
                                  semg_v[b]).wait()

        def writes(ci, b):
            dst = out_hbm.at[pl.ds(base + ci * c, c)]
            pltpu.async_copy(rows_s[b], dst.at[:, pl.ds(0, nw64)], semw_v[b])
            pltpu.async_copy(rows_r[b], dst.at[:, pl.ds(nw64, nw64)],
                             semw_v[b])

        def wait_writes(b):
            pltpu.make_async_copy(
                rows_s[b], out_hbm.at[pl.ds(0, c), pl.ds(0, nw64)],
                semw_v[b]).wait()
            pltpu.make_async_copy(
                rows_r[b], out_hbm.at[pl.ds(0, c), pl.ds(nw64, nw64)],
                semw_v[b]).wait()

        for b in range(_NSLOT):
            gathers(b, b)

        def ring_body(j, carry):
            for b in range(_NSLOT):
                ci = _NSLOT * j + b

                @pl.when(ci < full)
                def _():
                    wait_gathers(b)
                    writes(ci, b)

                    @pl.when(ci + _NSLOT < full)
                    def _():
                        wait_writes(b)
                        gathers(ci + _NSLOT, b)

            return carry

        lax.fori_loop(0, (full + _NSLOT - 1) // _NSLOT, ring_body, 0)

        for b in range(_NSLOT):
            wait_writes(b)

        if tail:
            toff = full * c
            rs_t = rows_s[0].at[pl.ds(0, tail)]
            rr_t = rows_r[0].at[pl.ds(0, tail)]
            pltpu.async_copy(
                ps_hbm.at[sidx_v.at[pl.ds(toff, tail)]], rs_t, semg_v[0])
            pltpu.async_copy(
                pr_hbm.at[ridx_v.at[pl.ds(toff, tail)]], rr_t, semg_v[0])
            pltpu.make_async_copy(ps_hbm.at[pl.ds(0, tail)], rs_t,
                                  semg_v[0]).wait()
            pltpu.make_async_copy(pr_hbm.at[pl.ds(0, tail)], rr_t,
                                  semg_v[0]).wait()
            tdst = out_hbm.at[pl.ds(base + toff, tail)]
            pltpu.sync_copy(rs_t, tdst.at[:, pl.ds(0, nw64)])
            pltpu.sync_copy(rr_t, tdst.at[:, pl.ds(nw64, nw64)])

    return k(senders, receivers, ps, pr)


_TC2_BLOCK = 10000


def _unpack_bf16(w32):
    lo = lax.bitcast_convert_type(w32 << 16, jnp.float32)
    hi = lax.bitcast_convert_type(w32 & jnp.int32(-65536), jnp.float32)
    return jnp.concatenate([lo, hi], axis=1)


def _tc2_body(g_ref, ed_ref, we_ref, b_ref, out_ref):
    prod = jnp.dot(ed_ref[...], we_ref[...], preferred_element_type=jnp.float32)
    g32 = g_ref[...]
    g = _unpack_bf16(g32[:, :D_FEAT // 2]) + _unpack_bf16(g32[:, D_FEAT // 2:])
    out_ref[...] = g + prod + b_ref[...]


def _edge_update(gathered, edata, we_t, b2d, ne):
    grid = ne // _TC2_BLOCK
    return pl.pallas_call(
        _tc2_body,
        grid=(grid,),
        in_specs=[
            pl.BlockSpec((_TC2_BLOCK, D_FEAT), lambda i: (i, 0)),
            pl.BlockSpec((_TC2_BLOCK, D_EDGE), lambda i: (i, 0)),
            pl.BlockSpec((D_EDGE, D_FEAT), lambda i: (0, 0)),
            pl.BlockSpec((1, D_FEAT), lambda i: (0, 0)),
        ],
        out_specs=pl.BlockSpec((_TC2_BLOCK, D_FEAT), lambda i: (i, 0)),
        out_shape=jax.ShapeDtypeStruct((ne, D_FEAT), jnp.float32),
    )(gathered, edata, we_t, b2d)


def kernel(vdata, edata, connectivity, W, b):
    senders = connectivity[0].astype(jnp.int32)
    receivers = connectivity[1].astype(jnp.int32)
    we_t = W[:, :D_EDGE].T
    ws_t = W[:, D_EDGE:D_EDGE + D_FEAT].T
    wr_t = W[:, D_EDGE + D_FEAT:].T
    b2d = b.reshape(1, D_FEAT)
    ps, pr = _node_projections(vdata, ws_t, wr_t)

    h = N_EDGES // _SPLITS
    outs = []
    for p in range(_SPLITS):
        sl = slice(p * h, (p + 1) * h)
        g = _sc_gather_sum(senders[sl], receivers[sl], ps, pr, h)
        outs.append(_edge_update(g, edata[sl], we_t, b2d, h))
    if _SPLITS == 1:
        return outs[0]
    return jnp.concatenate(outs, axis=0)
